# baseline (speedup 1.0000x reference)
import functools

import jax
import jax.numpy as jnp
from jax import lax
from jax.experimental import pallas as pl
from jax.experimental.pallas import tpu as pltpu
from jax.experimental.pallas import tpu_sc as plsc

_NC = 2
_NS = 16
_NW = _NC * _NS
_CHUNK = 128


def _merge_body(scale, main_ref, a_ref, bt_ref, out_ref):
    out_ref[...] = main_ref[...] + scale * jnp.dot(
        a_ref[...], bt_ref[...], preferred_element_type=jnp.float32
    )


def _merged_table(main_weight, lora_a, lora_bt, scale):
    v, d = main_weight.shape
    r = lora_a.shape[1]
    block = 1000
    grid = v // block
    return pl.pallas_call(
        functools.partial(_merge_body, scale),
        grid=(grid,),
        in_specs=[
            pl.BlockSpec((block, d), lambda i: (i, 0)),
            pl.BlockSpec((block, r), lambda i: (i, 0)),
            pl.BlockSpec((r, d), lambda i: (0, 0)),
        ],
        out_specs=pl.BlockSpec((block, d), lambda i: (i, 0)),
        out_shape=jax.ShapeDtypeStruct((v, d), jnp.float32),
    )(main_weight, lora_a, lora_bt)


def _make_gather(nchunk, d):
    n_per_w = nchunk * _CHUNK
    mesh = plsc.VectorSubcoreMesh(
        core_axis_name="c", subcore_axis_name="s", num_cores=_NC, num_subcores=_NS
    )

    @functools.partial(
        pl.kernel,
        out_type=jax.ShapeDtypeStruct((_NW * n_per_w, d), jnp.float32),
        mesh=mesh,
        scratch_types=[
            pltpu.VMEM((n_per_w,), jnp.int32),
            pltpu.VMEM((_CHUNK, d), jnp.float32),
            pltpu.SemaphoreType.DMA,
        ],
    )
    def gather(table_hbm, idx_hbm, out_hbm, idx_v, rows_v, sem):
        wid = lax.axis_index("s") * _NC + lax.axis_index("c")
        base = wid * n_per_w
        pltpu.sync_copy(idx_hbm.at[wid], idx_v)

        def chunk(j, carry):
            off = j * _CHUNK
            pltpu.async_copy(
                table_hbm.at[idx_v.at[pl.ds(off, _CHUNK)]], rows_v, sem
            ).wait()
            pltpu.sync_copy(rows_v, out_hbm.at[pl.ds(base + off, _CHUNK)])
            return carry

        lax.fori_loop(0, nchunk, chunk, 0)

    return gather


def _final_body(b_blk, l, m_ref, o_ref):
    o_ref[...] = m_ref[...].reshape(b_blk, l, m_ref.shape[-1])


def _final(rows, b, l):
    n, d = rows.shape
    b_blk = 16
    grid = b // b_blk
    return pl.pallas_call(
        functools.partial(_final_body, b_blk, l),
        grid=(grid,),
        in_specs=[pl.BlockSpec((b_blk * l, d), lambda i: (i, 0))],
        out_specs=pl.BlockSpec((b_blk, l, d), lambda i: (i, 0, 0)),
        out_shape=jax.ShapeDtypeStruct((b, l, d), jnp.float32),
    )(rows)


def kernel(idx, main_weight, lora_A, lora_B):
    b, l = idx.shape
    v, d = main_weight.shape
    rank = lora_A.shape[1]
    alpha = 32.0
    scale = alpha / rank

    merged = _merged_table(main_weight, lora_A, lora_B.T, scale)

    n = b * l
    assert n % (_NW * _CHUNK) == 0
    nchunk = n // (_NW * _CHUNK)
    idx2 = idx.astype(jnp.int32).reshape(_NW, nchunk * _CHUNK)
    rows = _make_gather(nchunk, d)(merged, idx2)
    return _final(rows, b, l)


# speedup vs baseline: 1.2981x; 1.2981x over previous
import functools

import jax
import jax.numpy as jnp
from jax import lax
from jax.experimental import pallas as pl
from jax.experimental.pallas import tpu as pltpu
from jax.experimental.pallas import tpu_sc as plsc

_NC = 2
_NS = 16
_NW = _NC * _NS
_CHUNK = 128


def _merge_body(scale, main_ref, a_ref, bt_ref, out_ref):
    out_ref[...] = main_ref[...] + scale * jnp.dot(
        a_ref[...], bt_ref[...], preferred_element_type=jnp.float32
    )


def _merged_table(main_weight, lora_a, lora_bt, scale):
    v, d = main_weight.shape
    r = lora_a.shape[1]
    block = 4000
    grid = v // block
    return pl.pallas_call(
        functools.partial(_merge_body, scale),
        grid=(grid,),
        in_specs=[
            pl.BlockSpec((block, d), lambda i: (i, 0)),
            pl.BlockSpec((block, r), lambda i: (i, 0)),
            pl.BlockSpec((r, d), lambda i: (0, 0)),
        ],
        out_specs=pl.BlockSpec((block, d), lambda i: (i, 0)),
        out_shape=jax.ShapeDtypeStruct((v, d), jnp.float32),
    )(main_weight, lora_a, lora_bt)


def _make_gather(nchunk, d):
    n_per_w = nchunk * _CHUNK
    assert nchunk % 2 == 0
    npairs = nchunk // 2
    mesh = plsc.VectorSubcoreMesh(
        core_axis_name="c", subcore_axis_name="s", num_cores=_NC, num_subcores=_NS
    )

    @functools.partial(
        pl.kernel,
        out_type=jax.ShapeDtypeStruct((_NW * n_per_w, d), jnp.float32),
        mesh=mesh,
        scratch_types=[
            pltpu.VMEM((n_per_w,), jnp.int32),
            pltpu.VMEM((2, _CHUNK, d), jnp.float32),
            pltpu.SemaphoreType.DMA,
        ],
    )
    def gather(table_hbm, idx_hbm, out_hbm, idx_v, rows_v, gsem):
        wid = lax.axis_index("s") * _NC + lax.axis_index("c")
        base = wid * n_per_w
        pltpu.sync_copy(idx_hbm.at[wid], idx_v)

        def fire(j, slot):
            pltpu.async_copy(
                table_hbm.at[idx_v.at[pl.ds(j * _CHUNK, _CHUNK)]],
                rows_v.at[slot],
                gsem,
            )

        def gwait(slot):
            pltpu.make_async_copy(
                table_hbm.at[idx_v.at[pl.ds(0, _CHUNK)]], rows_v.at[slot], gsem
            ).wait()

        def drain(j, slot):
            pltpu.sync_copy(rows_v.at[slot], out_hbm.at[pl.ds(base + j * _CHUNK, _CHUNK)])

        fire(0, 0)

        def pair(p, carry):
            j0 = 2 * p
            gwait(0)
            fire(j0 + 1, 1)
            drain(j0, 0)
            gwait(1)

            @pl.when(p + 1 < npairs)
            def _():
                fire(j0 + 2, 0)

            drain(j0 + 1, 1)
            return carry

        lax.fori_loop(0, npairs, pair, 0)

    return gather


def kernel(idx, main_weight, lora_A, lora_B):
    b, l = idx.shape
    v, d = main_weight.shape
    rank = lora_A.shape[1]
    alpha = 32.0
    scale = alpha / rank

    merged = _merged_table(main_weight, lora_A, lora_B.T, scale)

    n = b * l
    assert n % (_NW * _CHUNK) == 0
    nchunk = n // (_NW * _CHUNK)
    idx2 = idx.astype(jnp.int32).reshape(_NW, nchunk * _CHUNK)
    rows = _make_gather(nchunk, d)(merged, idx2)
    return rows.reshape(b, l, d)
